# baseline (speedup 1.0000x reference)
import functools

import jax
import jax.numpy as jnp
from jax import lax
from jax.experimental import pallas as pl
from jax.experimental.pallas import tpu as pltpu
from jax.experimental.pallas import tpu_sc as plsc

VOCAB = 100000
NODE_DIM = 128
MAX_BOND = 16

_ROWS_PER_BLOCK = 2048
_NC = 2
_NS = 16
_NW = _NC * _NS
_CHUNK = 128


def _proj_body(mt_ref, oht_ref, w_ref, b_ref, c_ref, p_ref):
    i = pl.program_id(0)
    rows = lax.broadcasted_iota(jnp.int32, (_ROWS_PER_BLOCK, 1), 0) + i * _ROWS_PER_BLOCK
    bias = jnp.where(rows < VOCAB, b_ref[...], 0.0)
    dn = (((0,), (0,)), ((), ()))
    c_ref[...] = lax.dot_general(mt_ref[...], w_ref[...], dn,
                                 preferred_element_type=jnp.float32) + bias

    @pl.when(i == 0)
    def _():
        p_ref[...] = lax.dot_general(oht_ref[...], w_ref[...], dn,
                                     preferred_element_type=jnp.float32)


def _project_tables(mt, oht, w, b2, cpad):
    grid = cpad // _ROWS_PER_BLOCK
    return pl.pallas_call(
        _proj_body,
        grid=(grid,),
        in_specs=[
            pl.BlockSpec((MAX_BOND, _ROWS_PER_BLOCK), lambda i: (0, i)),
            pl.BlockSpec((MAX_BOND, 24), lambda i: (0, 0)),
            pl.BlockSpec((MAX_BOND, NODE_DIM), lambda i: (0, 0)),
            pl.BlockSpec((1, NODE_DIM), lambda i: (0, 0)),
        ],
        out_specs=[
            pl.BlockSpec((_ROWS_PER_BLOCK, NODE_DIM), lambda i: (i, 0)),
            pl.BlockSpec((24, NODE_DIM), lambda i: (0, 0)),
        ],
        out_shape=[
            jax.ShapeDtypeStruct((cpad, NODE_DIM), jnp.float32),
            jax.ShapeDtypeStruct((24, NODE_DIM), jnp.float32),
        ],
    )(mt, oht, w, b2)


def _make_sc_kernel(n):
    per_w = n // _NW
    nchunks = per_w // _CHUNK
    mesh = plsc.VectorSubcoreMesh(core_axis_name="c", subcore_axis_name="s",
                                  num_cores=_NC, num_subcores=_NS)

    @functools.partial(
        pl.kernel,
        out_type=jax.ShapeDtypeStruct((n, NODE_DIM), jnp.float32),
        mesh=mesh,
        compiler_params=pltpu.CompilerParams(needs_layout_passes=False,
                                             use_tc_tiling_on_sc=True),
        scratch_types=[
            pltpu.VMEM((per_w,), jnp.int32),
            pltpu.VMEM((per_w,), jnp.int32),
            pltpu.VMEM((2, _CHUNK, NODE_DIM), jnp.float32),
            pltpu.VMEM((2, _CHUNK, NODE_DIM), jnp.float32),
            pltpu.VMEM((24 * NODE_DIM,), jnp.float32),
            pltpu.SemaphoreType.DMA,
            pltpu.SemaphoreType.DMA,
            pltpu.SemaphoreType.DMA,
            pltpu.SemaphoreType.DMA,
        ],
    )
    def sc_kernel(idx_hbm, rbp_hbm, emb_hbm, c_hbm, p_hbm, out_hbm,
                  idx_v, rbp_v, ebuf, wbuf, pbuf, sem_e, sem_w, sem_o0, sem_o1):
        wid = lax.axis_index("s") * _NC + lax.axis_index("c")
        w0 = wid * per_w
        sem_o = [sem_o0, sem_o1]
        c1 = pltpu.async_copy(idx_hbm.at[pl.ds(w0, per_w)], idx_v, sem_o0)
        c2 = pltpu.async_copy(rbp_hbm.at[pl.ds(w0, per_w)], rbp_v, sem_o1)
        c3 = pltpu.async_copy(p_hbm, pbuf, sem_e)
        c1.wait()
        c2.wait()
        c3.wait()
        cols = [lax.iota(jnp.int32, 16) + c * 16 for c in range(NODE_DIM // 16)]

        def stage(k, s):
            ii = idx_v.at[pl.ds(k * _CHUNK, _CHUNK)]
            pltpu.async_copy(emb_hbm.at[ii], ebuf.at[s], sem_e)
            pltpu.async_copy(c_hbm.at[ii], wbuf.at[s], sem_w)

        def wait_gathers(s):
            ii = idx_v.at[pl.ds(0, _CHUNK)]
            pltpu.make_async_copy(emb_hbm.at[ii], ebuf.at[s], sem_e).wait()
            pltpu.make_async_copy(c_hbm.at[ii], wbuf.at[s], sem_w).wait()

        def wait_writeback(s):
            pltpu.make_async_copy(
                wbuf.at[s], out_hbm.at[pl.ds(w0, _CHUNK)], sem_o[s]).wait()

        def compute(k, s):
            kc = k * _CHUNK

            @plsc.parallel_loop(0, _CHUNK, unroll=16)
            def _(t):
                rsp = plsc.load_gather(rbp_v, [jnp.full((16,), kc + t, jnp.int32)])
                rb = rsp * NODE_DIM
                for c in range(NODE_DIM // 16):
                    pv = plsc.load_gather(pbuf, [rb + cols[c]])
                    ev = ebuf[s, t, pl.ds(c * 16, 16)]
                    wv = wbuf[s, t, pl.ds(c * 16, 16)]
                    wbuf[s, t, pl.ds(c * 16, 16)] = ev * (wv + pv)

        stage(0, 0)

        @pl.loop(0, nchunks, step=2)
        def _(g):
            for b in range(2):
                k = g + b
                s = b
                o = 1 - b

                wait_gathers(s)

                @pl.when(k + 1 < nchunks)
                def _():
                    @pl.when(k >= 1)
                    def _():
                        wait_writeback(o)

                    stage(k + 1, o)

                compute(k, s)
                pltpu.async_copy(
                    wbuf.at[s], out_hbm.at[pl.ds(w0 + k * _CHUNK, _CHUNK)],
                    sem_o[s])

        wait_writeback((nchunks - 2) % 2)
        wait_writeback((nchunks - 1) % 2)

    return sc_kernel


def kernel(idx, root_bond_pos, embedding, bond_pos_tensors, one_hot_pos,
           W_root, b_root):
    b, l = idx.shape
    n = b * l
    idx_f = jnp.swapaxes(idx, 0, 1).reshape(n).astype(jnp.int32)
    rbp_f = jnp.swapaxes(root_bond_pos, 0, 1).reshape(n).astype(jnp.int32)

    cpad = ((bond_pos_tensors.shape[0] + _ROWS_PER_BLOCK - 1)
            // _ROWS_PER_BLOCK) * _ROWS_PER_BLOCK
    mt = bond_pos_tensors.T
    oht = jnp.pad(one_hot_pos.T, ((0, 0), (0, 24 - one_hot_pos.shape[0])))
    c, p = _project_tables(mt, oht, W_root.astype(jnp.float32),
                           b_root.reshape(1, NODE_DIM).astype(jnp.float32),
                           cpad)

    out = _make_sc_kernel(n)(idx_f, rbp_f, embedding, c, p.reshape(-1))
    return jnp.swapaxes(out.reshape(l, b, NODE_DIM), 0, 1)


# speedup vs baseline: 1.2415x; 1.2415x over previous
import functools

import jax
import jax.numpy as jnp
from jax import lax
from jax.experimental import pallas as pl
from jax.experimental.pallas import tpu as pltpu
from jax.experimental.pallas import tpu_sc as plsc

VOCAB = 100000
NODE_DIM = 128
MAX_BOND = 16

_ROWS_PER_BLOCK = 2048
_NC = 2
_NS = 16
_NW = _NC * _NS
_CHUNK = 128


def _proj_body(mt_ref, oht_ref, w_ref, b_ref, c_ref, p_ref):
    i = pl.program_id(0)
    rows = lax.broadcasted_iota(jnp.int32, (_ROWS_PER_BLOCK, 1), 0) + i * _ROWS_PER_BLOCK
    bias = jnp.where(rows < VOCAB, b_ref[...], 0.0)
    dn = (((0,), (0,)), ((), ()))
    c_ref[...] = lax.dot_general(mt_ref[...], w_ref[...], dn,
                                 preferred_element_type=jnp.float32) + bias

    @pl.when(i == 0)
    def _():
        p_ref[...] = lax.dot_general(oht_ref[...], w_ref[...], dn,
                                     preferred_element_type=jnp.float32)


def _project_tables(mt, oht, w, b2, cpad):
    grid = cpad // _ROWS_PER_BLOCK
    return pl.pallas_call(
        _proj_body,
        grid=(grid,),
        in_specs=[
            pl.BlockSpec((MAX_BOND, _ROWS_PER_BLOCK), lambda i: (0, i)),
            pl.BlockSpec((MAX_BOND, 24), lambda i: (0, 0)),
            pl.BlockSpec((MAX_BOND, NODE_DIM), lambda i: (0, 0)),
            pl.BlockSpec((1, NODE_DIM), lambda i: (0, 0)),
        ],
        out_specs=[
            pl.BlockSpec((_ROWS_PER_BLOCK, NODE_DIM), lambda i: (i, 0)),
            pl.BlockSpec((24, NODE_DIM), lambda i: (0, 0)),
        ],
        out_shape=[
            jax.ShapeDtypeStruct((cpad, NODE_DIM), jnp.float32),
            jax.ShapeDtypeStruct((24, NODE_DIM), jnp.float32),
        ],
    )(mt, oht, w, b2)


def _make_sc_kernel(n):
    per_w = n // _NW
    nchunks = per_w // _CHUNK
    mesh = plsc.VectorSubcoreMesh(core_axis_name="c", subcore_axis_name="s",
                                  num_cores=_NC, num_subcores=_NS)

    @functools.partial(
        pl.kernel,
        out_type=jax.ShapeDtypeStruct((n, NODE_DIM), jnp.float32),
        mesh=mesh,
        compiler_params=pltpu.CompilerParams(needs_layout_passes=False,
                                             use_tc_tiling_on_sc=True),
        scratch_types=[
            pltpu.VMEM((per_w,), jnp.int32),
            pltpu.VMEM((per_w,), jnp.int32),
            pltpu.VMEM((2, _CHUNK, NODE_DIM), jnp.float32),
            pltpu.VMEM((2, _CHUNK, NODE_DIM), jnp.float32),
            pltpu.VMEM((24 * NODE_DIM,), jnp.float32),
            pltpu.SemaphoreType.DMA,
            pltpu.SemaphoreType.DMA,
            pltpu.SemaphoreType.DMA,
            pltpu.SemaphoreType.DMA,
        ],
    )
    def sc_kernel(idx_hbm, rbp_hbm, emb_hbm, c_hbm, p_hbm, out_hbm,
                  idx_v, rbp_v, ebuf, wbuf, pbuf, sem_e, sem_w, sem_o0, sem_o1):
        wid = lax.axis_index("s") * _NC + lax.axis_index("c")
        w0 = wid * per_w
        sem_o = [sem_o0, sem_o1]
        c1 = pltpu.async_copy(idx_hbm.at[pl.ds(w0, per_w)], idx_v, sem_o0)
        c2 = pltpu.async_copy(rbp_hbm.at[pl.ds(w0, per_w)], rbp_v, sem_o1)
        c3 = pltpu.async_copy(p_hbm, pbuf, sem_e)
        c1.wait()
        c2.wait()
        c3.wait()
        cols = [lax.iota(jnp.int32, 16) + c * 16 for c in range(NODE_DIM // 16)]

        def stage(k, s):
            ii = idx_v.at[pl.ds(k * _CHUNK, _CHUNK)]
            pltpu.async_copy(emb_hbm.at[ii], ebuf.at[s], sem_e)
            pltpu.async_copy(c_hbm.at[ii], wbuf.at[s], sem_w)

        def wait_gathers(s):
            ii = idx_v.at[pl.ds(0, _CHUNK)]
            pltpu.make_async_copy(emb_hbm.at[ii], ebuf.at[s], sem_e).wait()
            pltpu.make_async_copy(c_hbm.at[ii], wbuf.at[s], sem_w).wait()

        def wait_writeback(s):
            pltpu.make_async_copy(
                wbuf.at[s], out_hbm.at[pl.ds(w0, _CHUNK)], sem_o[s]).wait()

        def compute(k, s):
            kc = k * _CHUNK

            @plsc.parallel_loop(0, _CHUNK, unroll=8)
            def _(t):
                rsp = plsc.load_gather(rbp_v, [jnp.full((16,), kc + t, jnp.int32)])
                rb = rsp * NODE_DIM
                for c in range(NODE_DIM // 16):
                    pv = plsc.load_gather(pbuf, [rb + cols[c]])
                    ev = ebuf[s, t, pl.ds(c * 16, 16)]
                    wv = wbuf[s, t, pl.ds(c * 16, 16)]
                    wbuf[s, t, pl.ds(c * 16, 16)] = ev * (wv + pv)

        stage(0, 0)

        @pl.loop(0, nchunks, step=2)
        def _(g):
            for b in range(2):
                k = g + b
                s = b
                o = 1 - b

                wait_gathers(s)

                @pl.when(k + 1 < nchunks)
                def _():
                    @pl.when(k >= 1)
                    def _():
                        wait_writeback(o)

                    stage(k + 1, o)

                compute(k, s)
                pltpu.async_copy(
                    wbuf.at[s], out_hbm.at[pl.ds(w0 + k * _CHUNK, _CHUNK)],
                    sem_o[s])

        wait_writeback((nchunks - 2) % 2)
        wait_writeback((nchunks - 1) % 2)

    return sc_kernel


def kernel(idx, root_bond_pos, embedding, bond_pos_tensors, one_hot_pos,
           W_root, b_root):
    b, l = idx.shape
    n = b * l
    idx_f = jnp.swapaxes(idx, 0, 1).reshape(n).astype(jnp.int32)
    rbp_f = jnp.swapaxes(root_bond_pos, 0, 1).reshape(n).astype(jnp.int32)

    cpad = ((bond_pos_tensors.shape[0] + _ROWS_PER_BLOCK - 1)
            // _ROWS_PER_BLOCK) * _ROWS_PER_BLOCK
    mt = bond_pos_tensors.T
    oht = jnp.pad(one_hot_pos.T, ((0, 0), (0, 24 - one_hot_pos.shape[0])))
    c, p = _project_tables(mt, oht, W_root.astype(jnp.float32),
                           b_root.reshape(1, NODE_DIM).astype(jnp.float32),
                           cpad)

    out = _make_sc_kernel(n)(idx_f, rbp_f, embedding, c, p.reshape(-1))
    return jnp.swapaxes(out.reshape(l, b, NODE_DIM), 0, 1)


# speedup vs baseline: 1.3109x; 1.0560x over previous
import functools

import jax
import jax.numpy as jnp
from jax import lax
from jax.experimental import pallas as pl
from jax.experimental.pallas import tpu as pltpu
from jax.experimental.pallas import tpu_sc as plsc

VOCAB = 100000
NODE_DIM = 128
MAX_BOND = 16

_ROWS_PER_BLOCK = 2048
_NC = 2
_NS = 16
_NW = _NC * _NS
_CHUNK = 128


def _pack_body(emb_ref, mt_ref, oht_ref, w_ref, b_ref, ec_ref, p_ref):
    i = pl.program_id(0)
    dn = (((0,), (0,)), ((), ()))
    c = lax.dot_general(mt_ref[...], w_ref[...], dn,
                        preferred_element_type=jnp.float32) + b_ref[...]
    eu = lax.bitcast_convert_type(emb_ref[...].astype(jnp.bfloat16),
                                  jnp.uint16).astype(jnp.uint32)
    cu = lax.bitcast_convert_type(c.astype(jnp.bfloat16),
                                  jnp.uint16).astype(jnp.uint32)
    ec_ref[...] = lax.bitcast_convert_type(eu | (cu << 16), jnp.float32)

    @pl.when(i == 0)
    def _():
        p_ref[...] = lax.dot_general(oht_ref[...], w_ref[...], dn,
                                     preferred_element_type=jnp.float32)


def _pack_tables(emb, mt, oht, w, b2, cpad):
    grid = cpad // _ROWS_PER_BLOCK
    return pl.pallas_call(
        _pack_body,
        grid=(grid,),
        in_specs=[
            pl.BlockSpec((_ROWS_PER_BLOCK, NODE_DIM), lambda i: (i, 0)),
            pl.BlockSpec((MAX_BOND, _ROWS_PER_BLOCK), lambda i: (0, i)),
            pl.BlockSpec((MAX_BOND, 24), lambda i: (0, 0)),
            pl.BlockSpec((MAX_BOND, NODE_DIM), lambda i: (0, 0)),
            pl.BlockSpec((1, NODE_DIM), lambda i: (0, 0)),
        ],
        out_specs=[
            pl.BlockSpec((_ROWS_PER_BLOCK, NODE_DIM), lambda i: (i, 0)),
            pl.BlockSpec((24, NODE_DIM), lambda i: (0, 0)),
        ],
        out_shape=[
            jax.ShapeDtypeStruct((cpad, NODE_DIM), jnp.float32),
            jax.ShapeDtypeStruct((24, NODE_DIM), jnp.float32),
        ],
    )(emb, mt, oht, w, b2)


def _make_sc_kernel(n):
    per_w = n // _NW
    nchunks = per_w // _CHUNK
    mesh = plsc.VectorSubcoreMesh(core_axis_name="c", subcore_axis_name="s",
                                  num_cores=_NC, num_subcores=_NS)

    @functools.partial(
        pl.kernel,
        out_type=jax.ShapeDtypeStruct((n, NODE_DIM), jnp.float32),
        mesh=mesh,
        compiler_params=pltpu.CompilerParams(needs_layout_passes=False,
                                             use_tc_tiling_on_sc=True),
        scratch_types=[
            pltpu.VMEM((per_w,), jnp.int32),
            pltpu.VMEM((per_w,), jnp.int32),
            pltpu.VMEM((2, _CHUNK, NODE_DIM), jnp.float32),
            pltpu.VMEM((2, _CHUNK, NODE_DIM), jnp.float32),
            pltpu.VMEM((24 * NODE_DIM,), jnp.float32),
            pltpu.SemaphoreType.DMA,
            pltpu.SemaphoreType.DMA,
            pltpu.SemaphoreType.DMA,
        ],
    )
    def sc_kernel(idx_hbm, rbp_hbm, ec_hbm, p_hbm, out_hbm,
                  idx_v, rbp_v, gbuf, obuf, pbuf, sem_g, sem_o0, sem_o1):
        wid = lax.axis_index("s") * _NC + lax.axis_index("c")
        w0 = wid * per_w
        sem_o = [sem_o0, sem_o1]
        c1 = pltpu.async_copy(idx_hbm.at[pl.ds(w0, per_w)], idx_v, sem_o0)
        c2 = pltpu.async_copy(rbp_hbm.at[pl.ds(w0, per_w)], rbp_v, sem_o1)
        c3 = pltpu.async_copy(p_hbm, pbuf, sem_g)
        c1.wait()
        c2.wait()
        c3.wait()
        cols = [lax.iota(jnp.int32, 16) + c * 16 for c in range(NODE_DIM // 16)]

        def stage(k, s):
            ii = idx_v.at[pl.ds(k * _CHUNK, _CHUNK)]
            pltpu.async_copy(ec_hbm.at[ii], gbuf.at[s], sem_g)

        def wait_gather(s):
            ii = idx_v.at[pl.ds(0, _CHUNK)]
            pltpu.make_async_copy(ec_hbm.at[ii], gbuf.at[s], sem_g).wait()

        def wait_writeback(s):
            pltpu.make_async_copy(
                obuf.at[s], out_hbm.at[pl.ds(w0, _CHUNK)], sem_o[s]).wait()

        def compute(k, s):
            kc = k * _CHUNK

            @plsc.parallel_loop(0, _CHUNK, unroll=8)
            def _(t):
                rsp = plsc.load_gather(rbp_v, [jnp.full((16,), kc + t, jnp.int32)])
                rb = rsp * NODE_DIM
                for g in range(NODE_DIM // 16):
                    ecw = gbuf[s, t, pl.ds(g * 16, 16)]
                    ev, cv = plsc.unpack(
                        plsc.bitcast(ecw, jnp.bfloat16),
                        format=plsc.PackFormat.INTERLEAVED,
                        preferred_element_type=jnp.float32)
                    pv = plsc.load_gather(pbuf, [rb + cols[g]])
                    obuf[s, t, pl.ds(g * 16, 16)] = ev * (cv + pv)

        stage(0, 0)

        @pl.loop(0, nchunks, step=2)
        def _(g):
            for b in range(2):
                k = g + b
                s = b
                o = 1 - b

                wait_gather(s)

                @pl.when(k + 1 < nchunks)
                def _():
                    @pl.when(k >= 1)
                    def _():
                        wait_writeback(o)

                    stage(k + 1, o)

                compute(k, s)
                pltpu.async_copy(
                    obuf.at[s], out_hbm.at[pl.ds(w0 + k * _CHUNK, _CHUNK)],
                    sem_o[s])

        wait_writeback((nchunks - 2) % 2)
        wait_writeback((nchunks - 1) % 2)

    return sc_kernel


def kernel(idx, root_bond_pos, embedding, bond_pos_tensors, one_hot_pos,
           W_root, b_root):
    b, l = idx.shape
    n = b * l
    idx_f = jnp.swapaxes(idx, 0, 1).reshape(n).astype(jnp.int32)
    rbp_f = jnp.swapaxes(root_bond_pos, 0, 1).reshape(n).astype(jnp.int32)

    cpad = ((bond_pos_tensors.shape[0] + _ROWS_PER_BLOCK - 1)
            // _ROWS_PER_BLOCK) * _ROWS_PER_BLOCK
    mt = bond_pos_tensors.T
    oht = jnp.pad(one_hot_pos.T, ((0, 0), (0, 24 - one_hot_pos.shape[0])))
    ec, p = _pack_tables(embedding, mt, oht, W_root.astype(jnp.float32),
                         b_root.reshape(1, NODE_DIM).astype(jnp.float32),
                         cpad)

    out = _make_sc_kernel(n)(idx_f, rbp_f, ec, p.reshape(-1))
    return jnp.swapaxes(out.reshape(l, b, NODE_DIM), 0, 1)


# speedup vs baseline: 1.4390x; 1.0977x over previous
import functools

import jax
import jax.numpy as jnp
from jax import lax
from jax.experimental import pallas as pl
from jax.experimental.pallas import tpu as pltpu
from jax.experimental.pallas import tpu_sc as plsc

VOCAB = 100000
NODE_DIM = 128
MAX_BOND = 16

_ROWS_PER_BLOCK = 4096
_NC = 2
_NS = 16
_NW = _NC * _NS
_CHUNK = 128


def _pack_body(emb_ref, mt_ref, oht_ref, w_ref, b_ref, ec_ref, p_ref):
    i = pl.program_id(0)
    dn = (((0,), (0,)), ((), ()))
    c = lax.dot_general(mt_ref[...], w_ref[...], dn,
                        preferred_element_type=jnp.float32) + b_ref[...]
    half = jnp.uint32(0x8000)
    eu = (lax.bitcast_convert_type(emb_ref[...], jnp.uint32) + half) >> 16
    cu = (lax.bitcast_convert_type(c, jnp.uint32) + half) & jnp.uint32(0xFFFF0000)
    ec_ref[...] = lax.bitcast_convert_type(eu | cu, jnp.float32)

    @pl.when(i == 0)
    def _():
        p_ref[...] = lax.dot_general(oht_ref[...], w_ref[...], dn,
                                     preferred_element_type=jnp.float32)


def _pack_tables(emb, mt, oht, w, b2, cpad):
    grid = cpad // _ROWS_PER_BLOCK
    return pl.pallas_call(
        _pack_body,
        grid=(grid,),
        in_specs=[
            pl.BlockSpec((_ROWS_PER_BLOCK, NODE_DIM), lambda i: (i, 0)),
            pl.BlockSpec((MAX_BOND, _ROWS_PER_BLOCK), lambda i: (0, i)),
            pl.BlockSpec((MAX_BOND, 24), lambda i: (0, 0)),
            pl.BlockSpec((MAX_BOND, NODE_DIM), lambda i: (0, 0)),
            pl.BlockSpec((1, NODE_DIM), lambda i: (0, 0)),
        ],
        out_specs=[
            pl.BlockSpec((_ROWS_PER_BLOCK, NODE_DIM), lambda i: (i, 0)),
            pl.BlockSpec((24, NODE_DIM), lambda i: (0, 0)),
        ],
        out_shape=[
            jax.ShapeDtypeStruct((cpad, NODE_DIM), jnp.float32),
            jax.ShapeDtypeStruct((24, NODE_DIM), jnp.float32),
        ],
    )(emb, mt, oht, w, b2)


def _make_sc_kernel(n):
    per_w = n // _NW
    nchunks = per_w // _CHUNK
    mesh = plsc.VectorSubcoreMesh(core_axis_name="c", subcore_axis_name="s",
                                  num_cores=_NC, num_subcores=_NS)

    @functools.partial(
        pl.kernel,
        out_type=jax.ShapeDtypeStruct((n, NODE_DIM), jnp.float32),
        mesh=mesh,
        compiler_params=pltpu.CompilerParams(needs_layout_passes=False,
                                             use_tc_tiling_on_sc=True),
        scratch_types=[
            pltpu.VMEM((per_w,), jnp.int32),
            pltpu.VMEM((per_w,), jnp.int32),
            pltpu.VMEM((2, _CHUNK, NODE_DIM), jnp.float32),
            pltpu.VMEM((2, _CHUNK, NODE_DIM), jnp.float32),
            pltpu.VMEM((24 * NODE_DIM,), jnp.float32),
            pltpu.SemaphoreType.DMA,
            pltpu.SemaphoreType.DMA,
            pltpu.SemaphoreType.DMA,
        ],
    )
    def sc_kernel(idx_hbm, rbp_hbm, ec_hbm, p_hbm, out_hbm,
                  idx_v, rbp_v, gbuf, obuf, pbuf, sem_g, sem_o0, sem_o1):
        wid = lax.axis_index("s") * _NC + lax.axis_index("c")
        w0 = wid * per_w
        sem_o = [sem_o0, sem_o1]
        c1 = pltpu.async_copy(idx_hbm.at[pl.ds(w0, per_w)], idx_v, sem_o0)
        c2 = pltpu.async_copy(rbp_hbm.at[pl.ds(w0, per_w)], rbp_v, sem_o1)
        c3 = pltpu.async_copy(p_hbm, pbuf, sem_g)
        c1.wait()
        c2.wait()
        c3.wait()
        cols = [lax.iota(jnp.int32, 16) + c * 16 for c in range(NODE_DIM // 16)]

        def stage(k, s):
            ii = idx_v.at[pl.ds(k * _CHUNK, _CHUNK)]
            pltpu.async_copy(ec_hbm.at[ii], gbuf.at[s], sem_g)

        def wait_gather(s):
            ii = idx_v.at[pl.ds(0, _CHUNK)]
            pltpu.make_async_copy(ec_hbm.at[ii], gbuf.at[s], sem_g).wait()

        def wait_writeback(s):
            pltpu.make_async_copy(
                obuf.at[s], out_hbm.at[pl.ds(w0, _CHUNK)], sem_o[s]).wait()

        def compute(k, s):
            kc = k * _CHUNK

            @plsc.parallel_loop(0, _CHUNK, unroll=8)
            def _(t):
                rsp = plsc.load_gather(rbp_v, [jnp.full((16,), kc + t, jnp.int32)])
                rb = rsp * NODE_DIM
                for g in range(NODE_DIM // 16):
                    ecw = gbuf[s, t, pl.ds(g * 16, 16)]
                    ev, cv = plsc.unpack(
                        plsc.bitcast(ecw, jnp.bfloat16),
                        format=plsc.PackFormat.INTERLEAVED,
                        preferred_element_type=jnp.float32)
                    pv = plsc.load_gather(pbuf, [rb + cols[g]])
                    obuf[s, t, pl.ds(g * 16, 16)] = ev * (cv + pv)

        stage(0, 0)

        @pl.loop(0, nchunks, step=2)
        def _(g):
            for b in range(2):
                k = g + b
                s = b
                o = 1 - b

                wait_gather(s)

                @pl.when(k + 1 < nchunks)
                def _():
                    @pl.when(k >= 1)
                    def _():
                        wait_writeback(o)

                    stage(k + 1, o)

                compute(k, s)
                pltpu.async_copy(
                    obuf.at[s], out_hbm.at[pl.ds(w0 + k * _CHUNK, _CHUNK)],
                    sem_o[s])

        wait_writeback((nchunks - 2) % 2)
        wait_writeback((nchunks - 1) % 2)

    return sc_kernel


def kernel(idx, root_bond_pos, embedding, bond_pos_tensors, one_hot_pos,
           W_root, b_root):
    b, l = idx.shape
    n = b * l
    idx_f = jnp.swapaxes(idx, 0, 1).reshape(n).astype(jnp.int32)
    rbp_f = jnp.swapaxes(root_bond_pos, 0, 1).reshape(n).astype(jnp.int32)

    cpad = ((bond_pos_tensors.shape[0] + _ROWS_PER_BLOCK - 1)
            // _ROWS_PER_BLOCK) * _ROWS_PER_BLOCK
    mt = bond_pos_tensors.T
    oht = jnp.pad(one_hot_pos.T, ((0, 0), (0, 24 - one_hot_pos.shape[0])))
    ec, p = _pack_tables(embedding, mt, oht, W_root.astype(jnp.float32),
                         b_root.reshape(1, NODE_DIM).astype(jnp.float32),
                         cpad)

    out = _make_sc_kernel(n)(idx_f, rbp_f, ec, p.reshape(-1))
    return jnp.swapaxes(out.reshape(l, b, NODE_DIM), 0, 1)
